# baseline (speedup 1.0000x reference)
import functools

import jax
import jax.numpy as jnp
from jax import lax
from jax.experimental import pallas as pl
from jax.experimental.pallas import tpu as pltpu
from jax.experimental.pallas import tpu_sc as plsc

VOCAB = 1000
VPAD = 1024
FEAT = 128
LPG = 128
K = 1
CHUNK = K * LPG
NBUF = 4


@functools.cache
def _build(B):
    info = plsc.get_sparse_core_info()
    NW = info.num_cores * info.num_subcores
    per_w = B // NW
    n_chunks = per_w // CHUNK
    assert per_w % CHUNK == 0 and n_chunks % NBUF == 0 and n_chunks >= 2 * NBUF
    rows_per_w = per_w // LPG

    mesh = plsc.VectorSubcoreMesh(core_axis_name="c", subcore_axis_name="s")

    @functools.partial(
        pl.kernel,
        out_type=jax.ShapeDtypeStruct((B, FEAT), jnp.float32),
        mesh=mesh,
        scratch_types=[
            pltpu.VMEM((NBUF, K, LPG), jnp.int32),
            pltpu.VMEM((NBUF, CHUNK, FEAT), jnp.float32),
            pltpu.VMEM_SHARED((VPAD, FEAT), jnp.float32),
        ] + [pltpu.SemaphoreType.DMA] * (2 * NBUF + 1),
    )
    def k(idx_hbm, table_hbm, out_hbm, idx_v, rows_v, table_sp, *sems):
        sem_i = sems[:NBUF]
        sem_o = sems[NBUF:2 * NBUF]
        sem_g = sems[2 * NBUF]
        wid = lax.axis_index("s") * info.num_cores + lax.axis_index("c")
        row0 = wid * rows_per_w
        out0 = wid * per_w

        def idx_copy(g, b):
            return pltpu.make_async_copy(
                idx_hbm.at[pl.ds(row0 + g * K, K)], idx_v.at[b], sem_i[b]
            )

        def out_copy(g, b):
            return pltpu.make_async_copy(
                rows_v.at[b], out_hbm.at[pl.ds(out0 + g * CHUNK, CHUNK)],
                sem_o[b],
            )

        sub = lax.axis_index("s")
        tslice = pl.ds(sub * (VPAD // 16), VPAD // 16)
        pltpu.sync_copy(table_hbm.at[tslice], rows_v.at[0, pl.ds(0, VPAD // 16)])
        pltpu.sync_copy(rows_v.at[0, pl.ds(0, VPAD // 16)], table_sp.at[tslice])
        plsc.subcore_barrier()

        for b in range(NBUF):
            idx_copy(b, b).start()

        @pl.loop(0, n_chunks, step=NBUF)
        def _(g0):
            for b in range(NBUF):
                g = g0 + b

                @pl.when(g >= NBUF)
                def _():
                    out_copy(g - NBUF, b).wait()

                idx_copy(g, b).wait()
                gathers = [
                    pltpu.async_copy(
                        table_sp.at[idx_v.at[b, j]],
                        rows_v.at[b, pl.ds(j * LPG, LPG)],
                        sem_g,
                    )
                    for j in range(K)
                ]
                for c in gathers:
                    c.wait()
                out_copy(g, b).start()

                @pl.when(g + NBUF < n_chunks)
                def _():
                    idx_copy(g + NBUF, b).start()

        for b in range(NBUF):
            out_copy(n_chunks - NBUF + b, b).wait()

    return k


def kernel(word_indexs, embedding_weight):
    B = word_indexs.shape[0] * word_indexs.shape[1]
    idx2d = word_indexs.reshape(B // LPG, LPG).astype(jnp.int32)
    tpad = jnp.pad(embedding_weight, ((0, VPAD - VOCAB), (0, 0)))
    out = _build(B)(idx2d, tpad)
    return out.reshape(word_indexs.shape[0], word_indexs.shape[1], FEAT)


# speedup vs baseline: 1.0419x; 1.0419x over previous
import functools

import jax
import jax.numpy as jnp
from jax import lax
from jax.experimental import pallas as pl
from jax.experimental.pallas import tpu as pltpu
from jax.experimental.pallas import tpu_sc as plsc

VOCAB = 1000
VPAD = 1024
FEAT = 128
LPG = 128
K = 2
CHUNK = K * LPG
NBUF = 3


@functools.cache
def _build(B):
    info = plsc.get_sparse_core_info()
    NW = info.num_cores * info.num_subcores
    per_w = B // NW
    n_chunks = per_w // CHUNK
    n_main = (n_chunks // NBUF) * NBUF
    assert per_w % CHUNK == 0 and n_chunks >= 2 * NBUF
    rows_per_w = per_w // LPG

    mesh = plsc.VectorSubcoreMesh(core_axis_name="c", subcore_axis_name="s")

    @functools.partial(
        pl.kernel,
        out_type=jax.ShapeDtypeStruct((B, FEAT), jnp.float32),
        mesh=mesh,
        scratch_types=[
            pltpu.VMEM((NBUF, K, LPG), jnp.int32),
            pltpu.VMEM((NBUF, CHUNK, FEAT), jnp.float32),
            pltpu.VMEM_SHARED((VPAD, FEAT), jnp.float32),
        ] + [pltpu.SemaphoreType.DMA] * (2 * NBUF + 1),
    )
    def k(idx_hbm, table_hbm, out_hbm, idx_v, rows_v, table_sp, *sems):
        sem_i = sems[:NBUF]
        sem_o = sems[NBUF:2 * NBUF]
        sem_g = sems[2 * NBUF]
        wid = lax.axis_index("s") * info.num_cores + lax.axis_index("c")
        row0 = wid * rows_per_w
        out0 = wid * per_w

        def idx_copy(g, b):
            return pltpu.make_async_copy(
                idx_hbm.at[pl.ds(row0 + g * K, K)], idx_v.at[b], sem_i[b]
            )

        def out_copy(g, b):
            return pltpu.make_async_copy(
                rows_v.at[b], out_hbm.at[pl.ds(out0 + g * CHUNK, CHUNK)],
                sem_o[b],
            )

        sub = lax.axis_index("s")
        tslice = pl.ds(sub * (VPAD // 16), VPAD // 16)
        pltpu.sync_copy(table_hbm.at[tslice], rows_v.at[0, pl.ds(0, VPAD // 16)])
        pltpu.sync_copy(rows_v.at[0, pl.ds(0, VPAD // 16)], table_sp.at[tslice])
        plsc.subcore_barrier()

        def chunk_step(g, b):
            @pl.when(g >= NBUF)
            def _():
                out_copy(g - NBUF, b).wait()

            idx_copy(g, b).wait()
            gathers = [
                pltpu.async_copy(
                    table_sp.at[idx_v.at[b, j]],
                    rows_v.at[b, pl.ds(j * LPG, LPG)],
                    sem_g,
                )
                for j in range(K)
            ]
            for c in gathers:
                c.wait()
            out_copy(g, b).start()

            @pl.when(g + NBUF < n_chunks)
            def _():
                idx_copy(g + NBUF, b).start()

        for b in range(NBUF):
            idx_copy(b, b).start()

        @pl.loop(0, n_main, step=NBUF)
        def _(g0):
            for b in range(NBUF):
                chunk_step(g0 + b, b)

        for g in range(n_main, n_chunks):
            chunk_step(jnp.int32(g), g % NBUF)

        for g in range(n_chunks - NBUF, n_chunks):
            out_copy(jnp.int32(g), g % NBUF).wait()

    return k


def kernel(word_indexs, embedding_weight):
    B = word_indexs.shape[0] * word_indexs.shape[1]
    idx2d = word_indexs.reshape(B // LPG, LPG).astype(jnp.int32)
    tpad = jnp.pad(embedding_weight, ((0, VPAD - VOCAB), (0, 0)))
    out = _build(B)(idx2d, tpad)
    return out.reshape(word_indexs.shape[0], word_indexs.shape[1], FEAT)


# speedup vs baseline: 1.0457x; 1.0036x over previous
import functools

import jax
import jax.numpy as jnp
from jax import lax
from jax.experimental import pallas as pl
from jax.experimental.pallas import tpu as pltpu
from jax.experimental.pallas import tpu_sc as plsc

VOCAB = 1000
VPAD = 1024
FEAT = 128
LPG = 128
K = 2
CHUNK = K * LPG
NBUF = 3


@functools.cache
def _build(B):
    info = plsc.get_sparse_core_info()
    NW = info.num_cores * info.num_subcores
    per_w = B // NW
    n_chunks = per_w // CHUNK
    n_main = (n_chunks // NBUF) * NBUF
    assert per_w % CHUNK == 0 and n_chunks >= 2 * NBUF
    rows_per_w = per_w // LPG

    mesh = plsc.VectorSubcoreMesh(core_axis_name="c", subcore_axis_name="s")

    @functools.partial(
        pl.kernel,
        out_type=jax.ShapeDtypeStruct((B, FEAT), jnp.float32),
        mesh=mesh,
        scratch_types=[
            pltpu.VMEM((NBUF, K, LPG), jnp.int32),
            pltpu.VMEM((NBUF, CHUNK, FEAT), jnp.float32),
            pltpu.VMEM_SHARED((VPAD, FEAT), jnp.float32),
        ] + [pltpu.SemaphoreType.DMA] * (3 * NBUF),
    )
    def k(idx_hbm, table_hbm, out_hbm, idx_v, rows_v, table_sp, *sems):
        sem_i = sems[:NBUF]
        sem_o = sems[NBUF:2 * NBUF]
        sem_gs = sems[2 * NBUF:]
        wid = lax.axis_index("s") * info.num_cores + lax.axis_index("c")
        row0 = wid * rows_per_w
        out0 = wid * per_w

        def idx_copy(g, b):
            return pltpu.make_async_copy(
                idx_hbm.at[pl.ds(row0 + g * K, K)], idx_v.at[b], sem_i[b]
            )

        def out_copy(g, b):
            return pltpu.make_async_copy(
                rows_v.at[b], out_hbm.at[pl.ds(out0 + g * CHUNK, CHUNK)],
                sem_o[b],
            )

        sub = lax.axis_index("s")
        tslice = pl.ds(sub * (VPAD // 16), VPAD // 16)
        pltpu.sync_copy(table_hbm.at[tslice], rows_v.at[0, pl.ds(0, VPAD // 16)])
        pltpu.sync_copy(rows_v.at[0, pl.ds(0, VPAD // 16)], table_sp.at[tslice])
        plsc.subcore_barrier()

        def gather(b, j):
            return pltpu.make_async_copy(
                table_sp.at[idx_v.at[b, j]],
                rows_v.at[b, pl.ds(j * LPG, LPG)],
                sem_gs[b],
            )

        def fire_gathers(g, b):
            @pl.when(g >= NBUF)
            def _():
                out_copy(g - NBUF, b).wait()

            idx_copy(g, b).wait()
            for j in range(K):
                gather(b, j).start()

        def chunk_step(g, b):
            for j in range(K):
                gather(b, j).wait()
            out_copy(g, b).start()

            @pl.when(g + NBUF < n_chunks)
            def _():
                idx_copy(g + NBUF, b).start()

            @pl.when(g + 1 < n_chunks)
            def _():
                fire_gathers(g + 1, (b + 1) % NBUF)

        for b in range(NBUF):
            idx_copy(b, b).start()
        fire_gathers(jnp.int32(0), 0)

        @pl.loop(0, n_main, step=NBUF)
        def _(g0):
            for b in range(NBUF):
                chunk_step(g0 + b, b)

        for g in range(n_main, n_chunks):
            chunk_step(jnp.int32(g), g % NBUF)

        for g in range(n_chunks - NBUF, n_chunks):
            out_copy(jnp.int32(g), g % NBUF).wait()

    return k


def kernel(word_indexs, embedding_weight):
    B = word_indexs.shape[0] * word_indexs.shape[1]
    idx2d = word_indexs.reshape(B // LPG, LPG).astype(jnp.int32)
    tpad = jnp.pad(embedding_weight, ((0, VPAD - VOCAB), (0, 0)))
    out = _build(B)(idx2d, tpad)
    return out.reshape(word_indexs.shape[0], word_indexs.shape[1], FEAT)


# speedup vs baseline: 1.0462x; 1.0005x over previous
import functools

import jax
import jax.numpy as jnp
from jax import lax
from jax.experimental import pallas as pl
from jax.experimental.pallas import tpu as pltpu
from jax.experimental.pallas import tpu_sc as plsc

VOCAB = 1000
VPAD = 1024
FEAT = 128
LPG = 128
K = 2
CHUNK = K * LPG
NBUF = 2


@functools.cache
def _build(B):
    info = plsc.get_sparse_core_info()
    NW = info.num_cores * info.num_subcores
    per_w = B // NW
    n_chunks = per_w // CHUNK
    assert per_w % CHUNK == 0 and n_chunks % 4 == 0 and n_chunks >= 8
    rows_per_w = per_w // LPG

    mesh = plsc.VectorSubcoreMesh(core_axis_name="c", subcore_axis_name="s")

    @functools.partial(
        pl.kernel,
        out_type=jax.ShapeDtypeStruct((B, FEAT), jnp.float32),
        mesh=mesh,
        scratch_types=[
            pltpu.VMEM((2, 2 * K, LPG), jnp.int32),
            pltpu.VMEM((NBUF, CHUNK, FEAT), jnp.float32),
            pltpu.VMEM_SHARED((VPAD, FEAT), jnp.float32),
        ] + [pltpu.SemaphoreType.DMA] * 6,
    )
    def k(idx_hbm, table_hbm, out_hbm, idx_v, rows_v, table_sp, *sems):
        sem_i = sems[:2]
        sem_o = sems[2:4]
        sem_gs = sems[4:]
        wid = lax.axis_index("s") * info.num_cores + lax.axis_index("c")
        row0 = wid * rows_per_w
        out0 = wid * per_w

        def idx_copy(m, p):
            return pltpu.make_async_copy(
                idx_hbm.at[pl.ds(row0 + m * 2 * K, 2 * K)], idx_v.at[p],
                sem_i[p],
            )

        def out_copy(g, b):
            return pltpu.make_async_copy(
                rows_v.at[b], out_hbm.at[pl.ds(out0 + g * CHUNK, CHUNK)],
                sem_o[b],
            )

        sub = lax.axis_index("s")
        tslice = pl.ds(sub * (VPAD // 16), VPAD // 16)
        pltpu.sync_copy(table_hbm.at[tslice], rows_v.at[0, pl.ds(0, VPAD // 16)])
        pltpu.sync_copy(rows_v.at[0, pl.ds(0, VPAD // 16)], table_sp.at[tslice])
        plsc.subcore_barrier()

        def gather(b, p, h, j):
            return pltpu.make_async_copy(
                table_sp.at[idx_v.at[p, h * K + j]],
                rows_v.at[b, pl.ds(j * LPG, LPG)],
                sem_gs[b],
            )

        def fire_gathers(g, b, p, h):
            @pl.when(g >= NBUF)
            def _():
                out_copy(g - NBUF, b).wait()

            if h == 0:
                idx_copy(g // 2, p).wait()
            for j in range(K):
                gather(b, p, h, j).start()

        def chunk_step(g, u):
            b, p, h = u % 2, (u // 2) % 2, u % 2
            for j in range(K):
                gather(b, p, h, j).wait()
            out_copy(g, b).start()

            if h == 1:
                @pl.when(g + 4 < n_chunks)
                def _():
                    idx_copy(g // 2 + 2, p).start()

            un = (u + 1) % 4
            @pl.when(g + 1 < n_chunks)
            def _():
                fire_gathers(g + 1, un % 2, (un // 2) % 2, un % 2)

        idx_copy(jnp.int32(0), 0).start()
        idx_copy(jnp.int32(1), 1).start()
        fire_gathers(jnp.int32(0), 0, 0, 0)

        @pl.loop(0, n_chunks, step=4)
        def _(g0):
            for u in range(4):
                chunk_step(g0 + u, u)

        for g in range(n_chunks - NBUF, n_chunks):
            out_copy(jnp.int32(g), g % NBUF).wait()

    return k


def kernel(word_indexs, embedding_weight):
    B = word_indexs.shape[0] * word_indexs.shape[1]
    idx2d = word_indexs.reshape(B // LPG, LPG).astype(jnp.int32)
    tpad = jnp.pad(embedding_weight, ((0, VPAD - VOCAB), (0, 0)))
    out = _build(B)(idx2d, tpad)
    return out.reshape(word_indexs.shape[0], word_indexs.shape[1], FEAT)
